# Initial kernel scaffold; baseline (speedup 1.0000x reference)
import jax
import jax.numpy as jnp
from jax.experimental import pallas as pl


def kernel(node_attr, edge_index, edge_attr, edge_sh, W1, b1, W2, b2, ln_w, ln_b):
    raise NotImplementedError("write your pallas kernel here")


# speedup vs baseline: 1.5827x; 1.5827x over previous
import functools

import jax
import jax.numpy as jnp
from jax import lax
from jax.experimental import pallas as pl
from jax.experimental.pallas import tpu as pltpu
from jax.experimental.pallas import tpu_sc as plsc

N_NODES = 10000
N_EDGES = 320000
F = 16
CH = 128
NCHUNK = 2560
E_PAD = NCHUNK * CH
NC = 2
NSUB = 16
CPW = NCHUNK // (NC * NSUB)
GROUP = 16
NZR = 624

_SC_PARAMS = pltpu.CompilerParams(use_tc_tiling_on_sc=False)


def _gather_body(dst_hbm, nattr_hbm, xg_hbm, idx_v, rows_v, sem):
    w = lax.axis_index("s") * NC + lax.axis_index("c")
    base = w * CPW
    pltpu.sync_copy(dst_hbm.at[pl.ds(base, CPW)], idx_v)
    for g in range(CPW // GROUP):
        cps = []
        for j in range(GROUP):
            t = g * GROUP + j
            cps.append(pltpu.async_copy(
                nattr_hbm.at[idx_v.at[t]], rows_v.at[pl.ds(j * CH, CH)], sem))
        for cp in cps:
            cp.wait()
        pltpu.sync_copy(
            rows_v,
            xg_hbm.at[pl.ds((base + g * GROUP) * CH, GROUP * CH), pl.ds(0, F)])


def _scatter_body(src_hbm, tpe_hbm, zer_hbm, acc_hbm,
                  idx_v, rows_v, zbuf_v, obuf_v, sem, shared):
    c = lax.axis_index("c")
    s = lax.axis_index("s")
    w = s * NC + c
    base = w * CPW
    pltpu.sync_copy(src_hbm.at[pl.ds(base, CPW)], idx_v)
    pltpu.sync_copy(zer_hbm, zbuf_v)
    pltpu.sync_copy(zbuf_v.at[pl.ds(0, NZR)], shared.at[pl.ds(s * NZR, NZR)])

    @pl.when(s == 0)
    def _():
        pltpu.sync_copy(zbuf_v.at[pl.ds(0, 16)],
                        shared.at[pl.ds(16 * NZR, 16)])

    plsc.subcore_barrier()

    def chunk(t, carry):
        cid = base + t
        pltpu.async_copy(
            tpe_hbm.at[pl.ds(cid * CH, CH), pl.ds(0, 2 * F)], rows_v,
            sem).wait()
        pltpu.async_copy(rows_v, shared.at[idx_v.at[t]], sem, add=True).wait()
        return carry

    lax.fori_loop(0, CPW, chunk, 0)
    plsc.subcore_barrier()
    pltpu.sync_copy(shared.at[pl.ds(s * NZR, NZR)], obuf_v)
    pltpu.sync_copy(obuf_v, acc_hbm.at[c, pl.ds(s * NZR, NZR)])

    @pl.when(s == 0)
    def _():
        pltpu.sync_copy(shared.at[pl.ds(16 * NZR, 16)],
                        obuf_v.at[pl.ds(0, 16)])
        pltpu.sync_copy(obuf_v.at[pl.ds(0, 16)],
                        acc_hbm.at[c, pl.ds(16 * NZR, 16)])


@functools.lru_cache(maxsize=None)
def _sc_calls():
    mesh = plsc.VectorSubcoreMesh(
        core_axis_name="c", subcore_axis_name="s",
        num_cores=NC, num_subcores=NSUB)
    gather_call = pl.kernel(
        _gather_body,
        out_type=jax.ShapeDtypeStruct((E_PAD, 128), jnp.float32),
        mesh=mesh,
        compiler_params=_SC_PARAMS,
        scratch_types=[
            pltpu.VMEM((CPW, CH), jnp.int32),
            pltpu.VMEM((GROUP * CH, F), jnp.float32),
            pltpu.SemaphoreType.DMA,
        ],
    )
    scatter_call = pl.kernel(
        _scatter_body,
        out_type=jax.ShapeDtypeStruct((NC, N_NODES, 2 * F), jnp.float32),
        mesh=mesh,
        compiler_params=_SC_PARAMS,
        scratch_types=[
            pltpu.VMEM((CPW, CH), jnp.int32),
            pltpu.VMEM((CH, 2 * F), jnp.float32),
            pltpu.VMEM((NZR, 2 * F), jnp.float32),
            pltpu.VMEM((NZR, 2 * F), jnp.float32),
            pltpu.SemaphoreType.DMA,
            pltpu.VMEM_SHARED((N_NODES, 2 * F), jnp.float32),
        ],
    )
    return gather_call, scatter_call


BLK = 2560
_NBLK_REAL = N_EDGES // BLK


def _dense_body(ea_ref, xg_ref, sh_ref, W1_ref, b1_ref, W2_ref, b2_ref, o_ref):
    ea = ea_ref[...]
    h = jnp.maximum(
        lax.dot_general(ea, W1_ref[...], (((1,), (0,)), ((), ())),
                        preferred_element_type=jnp.float32) + b1_ref[...], 0.0)
    w = lax.dot_general(h, W2_ref[...], (((1,), (0,)), ((), ())),
                        preferred_element_type=jnp.float32) + b2_ref[...]
    xs = xg_ref[:, 0:F] * sh_ref[...]
    acc = xs[:, 0:1] * w[:, 0:F]
    for i in range(1, F):
        acc = acc + xs[:, i:i + 1] * w[:, i * F:(i + 1) * F]
    tp = acc * 0.25
    ones = jnp.ones((BLK, 1), jnp.float32)
    zeros = jnp.zeros((BLK, 128 - F - 1), jnp.float32)
    row = jnp.concatenate([tp, ones, zeros], axis=1)
    gid = pl.program_id(0)
    valid = (gid * BLK + lax.broadcasted_iota(jnp.int32, (BLK, 1), 0)
             < N_EDGES)
    o_ref[...] = jnp.where(valid, row, 0.0)


def _clampi(i):
    return jnp.minimum(i, _NBLK_REAL - 1)


_dense_call = pl.pallas_call(
    _dense_body,
    grid=(E_PAD // BLK,),
    in_specs=[
        pl.BlockSpec((BLK, F), lambda i: (_clampi(i), 0)),
        pl.BlockSpec((BLK, 128), lambda i: (_clampi(i), 0)),
        pl.BlockSpec((BLK, 1), lambda i: (_clampi(i), 0)),
        pl.BlockSpec((F, F), lambda i: (0, 0)),
        pl.BlockSpec((1, F), lambda i: (0, 0)),
        pl.BlockSpec((F, F * F), lambda i: (0, 0)),
        pl.BlockSpec((1, F * F), lambda i: (0, 0)),
    ],
    out_specs=pl.BlockSpec((BLK, 128), lambda i: (i, 0)),
    out_shape=jax.ShapeDtypeStruct((E_PAD, 128), jnp.float32),
)


def _final_body(acc_ref, nat_ref, lnw_ref, lnb_ref, o_ref):
    a = acc_ref[0] + acc_ref[1]
    summed = a[:, 0:F]
    cnt = a[:, F:F + 1]
    out = summed / jnp.maximum(cnt, 1.0) + nat_ref[...]
    mean = jnp.sum(out, axis=1, keepdims=True) * (1.0 / F)
    xc = out - mean
    var = jnp.sum(xc * xc, axis=1, keepdims=True) * (1.0 / F)
    norm = lax.rsqrt(var + 1e-5)
    o_ref[...] = xc * (norm * lnw_ref[...]) + lnb_ref[...]


_final_call = pl.pallas_call(
    _final_body,
    out_shape=jax.ShapeDtypeStruct((N_NODES, F), jnp.float32),
)


def kernel(node_attr, edge_index, edge_attr, edge_sh, W1, b1, W2, b2, ln_w, ln_b):
    gather_call, scatter_call = _sc_calls()
    eip = jnp.pad(edge_index, ((0, 0), (0, E_PAD - N_EDGES)))
    dst2 = eip[1].reshape(NCHUNK, CH)
    src2 = eip[0].reshape(NCHUNK, CH)
    xg = gather_call(dst2, node_attr)
    tpe = _dense_call(edge_attr, xg, edge_sh, W1, b1.reshape(1, F),
                      W2, b2.reshape(1, F * F))
    zer = jnp.zeros((NZR, 2 * F), jnp.float32)
    acc = scatter_call(src2, tpe, zer)
    out = _final_call(acc, node_attr, ln_w.reshape(1, F), ln_b.reshape(1, F))
    return out


# speedup vs baseline: 4.8617x; 3.0717x over previous
import functools

import jax
import jax.numpy as jnp
from jax import lax
from jax.experimental import pallas as pl
from jax.experimental.pallas import tpu as pltpu
from jax.experimental.pallas import tpu_sc as plsc

N_NODES = 10000
N_EDGES = 320000
F = 16
CH = 128
NCHUNK = 2560
E_PAD = NCHUNK * CH
NC = 2
NSUB = 16
CPW = NCHUNK // (NC * NSUB)
GROUP = 16
NZR = 624

_SC_PARAMS = pltpu.CompilerParams(use_tc_tiling_on_sc=False)


def _gather_body(dst_hbm, nattr_hbm, xg_hbm, idx_v, rows_v, sem):
    w = lax.axis_index("s") * NC + lax.axis_index("c")
    base = w * CPW
    pltpu.sync_copy(dst_hbm.at[pl.ds(base, CPW)], idx_v)
    for g in range(CPW // GROUP):
        cps = []
        for j in range(GROUP):
            t = g * GROUP + j
            cps.append(pltpu.async_copy(
                nattr_hbm.at[idx_v.at[t]], rows_v.at[pl.ds(j * CH, CH)], sem))
        for cp in cps:
            cp.wait()
        pltpu.sync_copy(
            rows_v,
            xg_hbm.at[pl.ds((base + g * GROUP) * CH, GROUP * CH), pl.ds(0, F)])


def _scatter_body(src_hbm, tpe_hbm, zer_hbm, acc_hbm,
                  idx_v, rows_v, zbuf_v, obuf_v, sem, shared):
    c = lax.axis_index("c")
    s = lax.axis_index("s")
    w = s * NC + c
    base = w * CPW
    pltpu.sync_copy(src_hbm.at[pl.ds(base, CPW)], idx_v)
    pltpu.sync_copy(zer_hbm, zbuf_v)
    pltpu.sync_copy(zbuf_v.at[pl.ds(0, NZR)], shared.at[pl.ds(s * NZR, NZR)])

    @pl.when(s == 0)
    def _():
        pltpu.sync_copy(zbuf_v.at[pl.ds(0, 16)],
                        shared.at[pl.ds(16 * NZR, 16)])

    plsc.subcore_barrier()

    def chunk(t, carry):
        cid = base + t
        pltpu.async_copy(
            tpe_hbm.at[pl.ds(cid * CH, CH), pl.ds(0, 2 * F)], rows_v,
            sem).wait()
        pltpu.async_copy(rows_v, shared.at[idx_v.at[t]], sem, add=True).wait()
        return carry

    lax.fori_loop(0, CPW, chunk, 0)
    plsc.subcore_barrier()
    pltpu.sync_copy(shared.at[pl.ds(s * NZR, NZR)], obuf_v)
    pltpu.sync_copy(obuf_v, acc_hbm.at[c, pl.ds(s * NZR, NZR)])

    @pl.when(s == 0)
    def _():
        pltpu.sync_copy(shared.at[pl.ds(16 * NZR, 16)],
                        obuf_v.at[pl.ds(0, 16)])
        pltpu.sync_copy(obuf_v.at[pl.ds(0, 16)],
                        acc_hbm.at[c, pl.ds(16 * NZR, 16)])


@functools.lru_cache(maxsize=None)
def _sc_calls():
    mesh = plsc.VectorSubcoreMesh(
        core_axis_name="c", subcore_axis_name="s",
        num_cores=NC, num_subcores=NSUB)
    gather_call = pl.kernel(
        _gather_body,
        out_type=jax.ShapeDtypeStruct((E_PAD, 128), jnp.float32),
        mesh=mesh,
        compiler_params=_SC_PARAMS,
        scratch_types=[
            pltpu.VMEM((CPW, CH), jnp.int32),
            pltpu.VMEM((GROUP * CH, F), jnp.float32),
            pltpu.SemaphoreType.DMA,
        ],
    )
    scatter_call = pl.kernel(
        _scatter_body,
        out_type=jax.ShapeDtypeStruct((NC, N_NODES, 2 * F), jnp.float32),
        mesh=mesh,
        compiler_params=_SC_PARAMS,
        scratch_types=[
            pltpu.VMEM((CPW, CH), jnp.int32),
            pltpu.VMEM((CH, 2 * F), jnp.float32),
            pltpu.VMEM((NZR, 2 * F), jnp.float32),
            pltpu.VMEM((NZR, 2 * F), jnp.float32),
            pltpu.SemaphoreType.DMA,
            pltpu.VMEM_SHARED((N_NODES, 2 * F), jnp.float32),
        ],
    )
    return gather_call, scatter_call


BLK = 2560
_NBLK_REAL = N_EDGES // BLK

import numpy as _np

_R_np = _np.kron(_np.eye(F, dtype=_np.float32), _np.ones((1, F), _np.float32))
_T_np = _np.kron(_np.ones((1, F), _np.float32), _np.eye(F, dtype=_np.float32))


def _mm(a, b):
    return lax.dot_general(a, b, (((1,), (0,)), ((), ())),
                           preferred_element_type=jnp.float32)


def _dense_body(ea_ref, xg_ref, sh_ref, W1_ref, b1_ref, R_ref, T_ref, V_ref,
                B2_ref, o_ref):
    ea = ea_ref[...]
    h = jnp.maximum(_mm(ea, W1_ref[...]) + b1_ref[...], 0.0)
    xs = xg_ref[:, 0:F] * sh_ref[...]
    z = _mm(h, R_ref[...]) * _mm(xs, T_ref[...])
    tp = (_mm(z, V_ref[...]) + _mm(xs, B2_ref[...])) * 0.25
    gid = pl.program_id(0)
    validf = ((gid * BLK + lax.broadcasted_iota(jnp.int32, (BLK, 1), 0)
               < N_EDGES)).astype(jnp.float32)
    col0 = (lax.broadcasted_iota(jnp.int32, (BLK, F), 1) == 0)
    o_ref[:, 0:F] = tp * validf
    o_ref[:, F:2 * F] = validf * col0.astype(jnp.float32)
    o_ref[:, 2 * F:128] = jnp.zeros((BLK, 128 - 2 * F), jnp.float32)


def _clampi(i):
    return jnp.minimum(i, _NBLK_REAL - 1)


_dense_call = pl.pallas_call(
    _dense_body,
    grid=(E_PAD // BLK,),
    in_specs=[
        pl.BlockSpec((BLK, F), lambda i: (_clampi(i), 0)),
        pl.BlockSpec((BLK, 128), lambda i: (_clampi(i), 0)),
        pl.BlockSpec((BLK, 1), lambda i: (_clampi(i), 0)),
        pl.BlockSpec((F, F), lambda i: (0, 0)),
        pl.BlockSpec((1, F), lambda i: (0, 0)),
        pl.BlockSpec((F, F * F), lambda i: (0, 0)),
        pl.BlockSpec((F, F * F), lambda i: (0, 0)),
        pl.BlockSpec((F * F, F), lambda i: (0, 0)),
        pl.BlockSpec((F, F), lambda i: (0, 0)),
    ],
    out_specs=pl.BlockSpec((BLK, 128), lambda i: (i, 0)),
    out_shape=jax.ShapeDtypeStruct((E_PAD, 128), jnp.float32),
)


def _final_body(acc_ref, nat_ref, lnw_ref, lnb_ref, o_ref):
    a = acc_ref[0] + acc_ref[1]
    summed = a[:, 0:F]
    cnt = a[:, F:F + 1]
    out = summed / jnp.maximum(cnt, 1.0) + nat_ref[...]
    mean = jnp.sum(out, axis=1, keepdims=True) * (1.0 / F)
    xc = out - mean
    var = jnp.sum(xc * xc, axis=1, keepdims=True) * (1.0 / F)
    norm = lax.rsqrt(var + 1e-5)
    o_ref[...] = xc * (norm * lnw_ref[...]) + lnb_ref[...]


_final_call = pl.pallas_call(
    _final_body,
    out_shape=jax.ShapeDtypeStruct((N_NODES, F), jnp.float32),
)


def kernel(node_attr, edge_index, edge_attr, edge_sh, W1, b1, W2, b2, ln_w, ln_b):
    gather_call, scatter_call = _sc_calls()
    eip = jnp.pad(edge_index, ((0, 0), (0, E_PAD - N_EDGES)))
    dst2 = eip[1].reshape(NCHUNK, CH)
    src2 = eip[0].reshape(NCHUNK, CH)
    xg = gather_call(dst2, node_attr)
    V = W2.reshape(F, F, F).reshape(F * F, F)
    B2 = b2.reshape(F, F)
    tpe = _dense_call(edge_attr, xg, edge_sh, W1, b1.reshape(1, F),
                      jnp.asarray(_R_np), jnp.asarray(_T_np), V, B2)
    zer = jnp.zeros((NZR, 2 * F), jnp.float32)
    acc = scatter_call(src2, tpe, zer)
    out = _final_call(acc, node_attr, ln_w.reshape(1, F), ln_b.reshape(1, F))
    return out


# speedup vs baseline: 4.9494x; 1.0180x over previous
import functools

import jax
import jax.numpy as jnp
from jax import lax
from jax.experimental import pallas as pl
from jax.experimental.pallas import tpu as pltpu
from jax.experimental.pallas import tpu_sc as plsc

N_NODES = 10000
N_EDGES = 320000
F = 16
CH = 128
NCHUNK = 2560
E_PAD = NCHUNK * CH
NC = 2
NSUB = 16
CPW = NCHUNK // (NC * NSUB)
GROUP = 16
NZR = 624

_SC_PARAMS = pltpu.CompilerParams(use_tc_tiling_on_sc=False)


def _gather_body(dst_hbm, nattr_hbm, xg_hbm, idx_v, rows_v, sem_g, sem_c):
    w = lax.axis_index("s") * NC + lax.axis_index("c")
    base = w * CPW
    pltpu.sync_copy(dst_hbm.at[pl.ds(base, CPW)], idx_v)
    ng = CPW // GROUP
    outs = [None] * ng
    for g in range(ng):
        b = g % 2
        if g >= 2:
            outs[g - 2].wait()
        cps = []
        for j in range(GROUP):
            t = g * GROUP + j
            cps.append(pltpu.async_copy(
                nattr_hbm.at[idx_v.at[t]], rows_v.at[b, pl.ds(j * CH, CH)],
                sem_g))
        for cp in cps:
            cp.wait()
        outs[g] = pltpu.async_copy(
            rows_v.at[b],
            xg_hbm.at[pl.ds((base + g * GROUP) * CH, GROUP * CH),
                      pl.ds(0, F)],
            sem_c)
    outs[ng - 2].wait()
    outs[ng - 1].wait()


def _scatter_body(src_hbm, tpe_hbm, zer_hbm, acc_hbm,
                  idx_v, rows_v, zbuf_v, obuf_v, sem_l, sem_a, shared):
    c = lax.axis_index("c")
    s = lax.axis_index("s")
    w = s * NC + c
    base = w * CPW
    pltpu.sync_copy(src_hbm.at[pl.ds(base, CPW)], idx_v)
    pltpu.sync_copy(zer_hbm, zbuf_v)
    pltpu.sync_copy(zbuf_v.at[pl.ds(0, NZR)], shared.at[pl.ds(s * NZR, NZR)])

    @pl.when(s == 0)
    def _():
        pltpu.sync_copy(zbuf_v.at[pl.ds(0, 16)],
                        shared.at[pl.ds(16 * NZR, 16)])

    plsc.subcore_barrier()

    def load(t, b):
        return pltpu.async_copy(
            tpe_hbm.at[pl.ds((base + t) * CH, CH), pl.ds(0, 2 * F)],
            rows_v.at[b], sem_l)

    lds = [None, None]
    adds = [None, None]
    lds[0] = load(0, 0)
    for t in range(CPW):
        b = t % 2
        lds[b].wait()
        if adds[1 - b] is not None:
            adds[1 - b].wait()
        if t + 1 < CPW:
            lds[1 - b] = load(t + 1, 1 - b)
        adds[b] = pltpu.async_copy(rows_v.at[b], shared.at[idx_v.at[t]],
                                   sem_a, add=True)
    adds[(CPW - 1) % 2].wait()
    plsc.subcore_barrier()
    pltpu.sync_copy(shared.at[pl.ds(s * NZR, NZR)], obuf_v)
    pltpu.sync_copy(obuf_v, acc_hbm.at[c, pl.ds(s * NZR, NZR)])

    @pl.when(s == 0)
    def _():
        pltpu.sync_copy(shared.at[pl.ds(16 * NZR, 16)],
                        obuf_v.at[pl.ds(0, 16)])
        pltpu.sync_copy(obuf_v.at[pl.ds(0, 16)],
                        acc_hbm.at[c, pl.ds(16 * NZR, 16)])


@functools.lru_cache(maxsize=None)
def _sc_calls():
    mesh = plsc.VectorSubcoreMesh(
        core_axis_name="c", subcore_axis_name="s",
        num_cores=NC, num_subcores=NSUB)
    gather_call = pl.kernel(
        _gather_body,
        out_type=jax.ShapeDtypeStruct((E_PAD, 128), jnp.float32),
        mesh=mesh,
        compiler_params=_SC_PARAMS,
        scratch_types=[
            pltpu.VMEM((CPW, CH), jnp.int32),
            pltpu.VMEM((2, GROUP * CH, F), jnp.float32),
            pltpu.SemaphoreType.DMA,
            pltpu.SemaphoreType.DMA,
        ],
    )
    scatter_call = pl.kernel(
        _scatter_body,
        out_type=jax.ShapeDtypeStruct((NC, N_NODES, 2 * F), jnp.float32),
        mesh=mesh,
        compiler_params=_SC_PARAMS,
        scratch_types=[
            pltpu.VMEM((CPW, CH), jnp.int32),
            pltpu.VMEM((2, CH, 2 * F), jnp.float32),
            pltpu.VMEM((NZR, 2 * F), jnp.float32),
            pltpu.VMEM((NZR, 2 * F), jnp.float32),
            pltpu.SemaphoreType.DMA,
            pltpu.SemaphoreType.DMA,
            pltpu.VMEM_SHARED((N_NODES, 2 * F), jnp.float32),
        ],
    )
    return gather_call, scatter_call


BLK = 2560
_NBLK_REAL = N_EDGES // BLK

import numpy as _np

_R_np = _np.kron(_np.eye(F, dtype=_np.float32), _np.ones((1, F), _np.float32))
_T_np = _np.kron(_np.ones((1, F), _np.float32), _np.eye(F, dtype=_np.float32))


def _mm(a, b):
    return lax.dot_general(a, b, (((1,), (0,)), ((), ())),
                           preferred_element_type=jnp.float32)


def _dense_body(ea_ref, xg_ref, sh_ref, W1_ref, b1_ref, R_ref, T_ref, V_ref,
                B2_ref, o_ref):
    i = pl.program_id(0)
    ea = ea_ref[...]
    h = jnp.maximum(_mm(ea, W1_ref[...]) + b1_ref[...], 0.0)
    xs = xg_ref[:, 0:F] * sh_ref[...]
    z = _mm(h, R_ref[...]) * _mm(xs, T_ref[...])
    tp = (_mm(z, V_ref[...]) + _mm(xs, B2_ref[...])) * 0.25
    validf = ((i * BLK + lax.broadcasted_iota(jnp.int32, (BLK, 1), 0)
               < N_EDGES)).astype(jnp.float32)
    col0 = (lax.broadcasted_iota(jnp.int32, (BLK, F), 1) == 0)
    o_ref[:, 0:F] = tp * validf
    o_ref[:, F:2 * F] = validf * col0.astype(jnp.float32)
    o_ref[:, 2 * F:128] = jnp.zeros((BLK, 128 - 2 * F), jnp.float32)


def _clampi(i):
    return jnp.minimum(i, _NBLK_REAL - 1)


_dense_call = pl.pallas_call(
    _dense_body,
    grid=(E_PAD // BLK,),
    in_specs=[
        pl.BlockSpec((BLK, F), lambda i: (_clampi(i), 0)),
        pl.BlockSpec((BLK, 128), lambda i: (_clampi(i), 0)),
        pl.BlockSpec((BLK, 1), lambda i: (_clampi(i), 0)),
        pl.BlockSpec((F, F), lambda i: (0, 0)),
        pl.BlockSpec((1, F), lambda i: (0, 0)),
        pl.BlockSpec((F, F * F), lambda i: (0, 0)),
        pl.BlockSpec((F, F * F), lambda i: (0, 0)),
        pl.BlockSpec((F * F, F), lambda i: (0, 0)),
        pl.BlockSpec((F, F), lambda i: (0, 0)),
    ],
    out_specs=pl.BlockSpec((BLK, 128), lambda i: (i, 0)),
    out_shape=jax.ShapeDtypeStruct((E_PAD, 128), jnp.float32),
)


def _final_body(acc_ref, nat_ref, lnw_ref, lnb_ref, o_ref):
    a = acc_ref[0] + acc_ref[1]
    summed = a[:, 0:F]
    cnt = a[:, F:F + 1]
    out = summed / jnp.maximum(cnt, 1.0) + nat_ref[...]
    mean = jnp.sum(out, axis=1, keepdims=True) * (1.0 / F)
    xc = out - mean
    var = jnp.sum(xc * xc, axis=1, keepdims=True) * (1.0 / F)
    norm = lax.rsqrt(var + 1e-5)
    o_ref[...] = xc * (norm * lnw_ref[...]) + lnb_ref[...]


_final_call = pl.pallas_call(
    _final_body,
    out_shape=jax.ShapeDtypeStruct((N_NODES, F), jnp.float32),
)


def kernel(node_attr, edge_index, edge_attr, edge_sh, W1, b1, W2, b2, ln_w, ln_b):
    gather_call, scatter_call = _sc_calls()
    eip = jnp.pad(edge_index, ((0, 0), (0, E_PAD - N_EDGES)))
    dst2 = eip[1].reshape(NCHUNK, CH)
    src2 = eip[0].reshape(NCHUNK, CH)
    xg = gather_call(dst2, node_attr)
    V = W2.reshape(F, F, F).reshape(F * F, F)
    B2 = b2.reshape(F, F)
    tpe = _dense_call(edge_attr, xg, edge_sh, W1, b1.reshape(1, F),
                      jnp.asarray(_R_np), jnp.asarray(_T_np), V, B2)
    zer = jnp.zeros((NZR, 2 * F), jnp.float32)
    acc = scatter_call(src2, tpe, zer)
    out = _final_call(acc, node_attr, ln_w.reshape(1, F), ln_b.reshape(1, F))
    return out


# speedup vs baseline: 5.8120x; 1.1743x over previous
import functools

import jax
import jax.numpy as jnp
from jax import lax
from jax.experimental import pallas as pl
from jax.experimental.pallas import tpu as pltpu
from jax.experimental.pallas import tpu_sc as plsc

N_NODES = 10000
N_EDGES = 320000
F = 16
CH = 128
NCHUNK = 2560
E_PAD = NCHUNK * CH
NC = 2
NSUB = 16
CPW = NCHUNK // (NC * NSUB)
GROUP = 16
NZR = 624

_SC_PARAMS = pltpu.CompilerParams(use_tc_tiling_on_sc=False)


def _gather_body(dst_hbm, nattr_hbm, xg_hbm, idx_v, rows_v, sem_g, sem_c):
    w = lax.axis_index("s") * NC + lax.axis_index("c")
    base = w * CPW
    pltpu.sync_copy(dst_hbm.at[pl.ds(base, CPW)], idx_v)
    ng = CPW // GROUP
    outs = [None] * ng
    for g in range(ng):
        b = g % 2
        if g >= 2:
            outs[g - 2].wait()
        cps = []
        for j in range(GROUP):
            t = g * GROUP + j
            cps.append(pltpu.async_copy(
                nattr_hbm.at[idx_v.at[t]], rows_v.at[b, pl.ds(j * CH, CH)],
                sem_g))
        for cp in cps:
            cp.wait()
        outs[g] = pltpu.async_copy(
            rows_v.at[b],
            xg_hbm.at[pl.ds((base + g * GROUP) * CH, GROUP * CH),
                      pl.ds(0, F)],
            sem_c)
    outs[ng - 2].wait()
    outs[ng - 1].wait()


def _scatter_body(src_hbm, tpe_hbm, zer_hbm, acc_hbm,
                  idx_v, rows_v, zbuf_v, obuf_v, sem_l, sem_a, shared):
    c = lax.axis_index("c")
    s = lax.axis_index("s")
    w = s * NC + c
    base = w * CPW
    pltpu.sync_copy(src_hbm.at[pl.ds(base, CPW)], idx_v)
    pltpu.sync_copy(zer_hbm, zbuf_v)
    pltpu.sync_copy(zbuf_v.at[pl.ds(0, NZR)], shared.at[pl.ds(s * NZR, NZR)])

    @pl.when(s == 0)
    def _():
        pltpu.sync_copy(zbuf_v.at[pl.ds(0, 16)],
                        shared.at[pl.ds(16 * NZR, 16)])

    plsc.subcore_barrier()

    def load(t, b):
        return pltpu.async_copy(
            tpe_hbm.at[pl.ds((base + t) * CH, CH), pl.ds(0, 2 * F)],
            rows_v.at[b], sem_l)

    lds = [None, None]
    adds = [None, None]
    lds[0] = load(0, 0)
    for t in range(CPW):
        b = t % 2
        lds[b].wait()
        if adds[1 - b] is not None:
            adds[1 - b].wait()
        if t + 1 < CPW:
            lds[1 - b] = load(t + 1, 1 - b)
        adds[b] = pltpu.async_copy(rows_v.at[b], shared.at[idx_v.at[t]],
                                   sem_a, add=True)
    adds[(CPW - 1) % 2].wait()
    plsc.subcore_barrier()
    pltpu.sync_copy(shared.at[pl.ds(s * NZR, NZR)], obuf_v)
    pltpu.sync_copy(obuf_v, acc_hbm.at[c, pl.ds(s * NZR, NZR)])

    @pl.when(s == 0)
    def _():
        pltpu.sync_copy(shared.at[pl.ds(16 * NZR, 16)],
                        obuf_v.at[pl.ds(0, 16)])
        pltpu.sync_copy(obuf_v.at[pl.ds(0, 16)],
                        acc_hbm.at[c, pl.ds(16 * NZR, 16)])


@functools.lru_cache(maxsize=None)
def _sc_calls():
    mesh = plsc.VectorSubcoreMesh(
        core_axis_name="c", subcore_axis_name="s",
        num_cores=NC, num_subcores=NSUB)
    gather_call = pl.kernel(
        _gather_body,
        out_type=jax.ShapeDtypeStruct((E_PAD, 128), jnp.float32),
        mesh=mesh,
        compiler_params=_SC_PARAMS,
        scratch_types=[
            pltpu.VMEM((CPW, CH), jnp.int32),
            pltpu.VMEM((2, GROUP * CH, F), jnp.float32),
            pltpu.SemaphoreType.DMA,
            pltpu.SemaphoreType.DMA,
        ],
    )
    scatter_call = pl.kernel(
        _scatter_body,
        out_type=jax.ShapeDtypeStruct((NC, N_NODES, 2 * F), jnp.float32),
        mesh=mesh,
        compiler_params=_SC_PARAMS,
        scratch_types=[
            pltpu.VMEM((CPW, CH), jnp.int32),
            pltpu.VMEM((2, CH, 2 * F), jnp.float32),
            pltpu.VMEM((NZR, 2 * F), jnp.float32),
            pltpu.VMEM((NZR, 2 * F), jnp.float32),
            pltpu.SemaphoreType.DMA,
            pltpu.SemaphoreType.DMA,
            pltpu.VMEM_SHARED((N_NODES, 2 * F), jnp.float32),
        ],
    )
    return gather_call, scatter_call


BLK = 2560
_NBLK_REAL = N_EDGES // BLK

import numpy as _np

_R_np = _np.kron(_np.eye(F, dtype=_np.float32), _np.ones((1, F), _np.float32))
_T_np = _np.kron(_np.ones((1, F), _np.float32), _np.eye(F, dtype=_np.float32))


def _mm(a, b):
    return lax.dot_general(a, b, (((1,), (0,)), ((), ())),
                           preferred_element_type=jnp.float32)


def _dense_body(ea_ref, xg_ref, sh_ref, W1_ref, b1_ref, R_ref, T_ref, V_ref,
                B2_ref, o_ref):
    i = pl.program_id(0)
    ea = jnp.transpose(ea_ref[...], (1, 0))
    sh = jnp.transpose(sh_ref[...], (1, 0))
    h = jnp.maximum(_mm(ea, W1_ref[...]) + b1_ref[...], 0.0)
    xs = xg_ref[:, 0:F] * sh
    z = _mm(h, R_ref[...]) * _mm(xs, T_ref[...])
    tp = (_mm(z, V_ref[...]) + _mm(xs, B2_ref[...])) * 0.25
    validf = ((i * BLK + lax.broadcasted_iota(jnp.int32, (BLK, 1), 0)
               < N_EDGES)).astype(jnp.float32)
    col0 = (lax.broadcasted_iota(jnp.int32, (BLK, F), 1) == 0)
    o_ref[:, 0:F] = tp * validf
    o_ref[:, F:2 * F] = validf * col0.astype(jnp.float32)
    o_ref[:, 2 * F:128] = jnp.zeros((BLK, 128 - 2 * F), jnp.float32)


def _clampi(i):
    return jnp.minimum(i, _NBLK_REAL - 1)


_dense_call = pl.pallas_call(
    _dense_body,
    grid=(E_PAD // BLK,),
    in_specs=[
        pl.BlockSpec((F, BLK), lambda i: (0, _clampi(i))),
        pl.BlockSpec((BLK, 128), lambda i: (_clampi(i), 0)),
        pl.BlockSpec((1, BLK), lambda i: (0, _clampi(i))),
        pl.BlockSpec((F, F), lambda i: (0, 0)),
        pl.BlockSpec((1, F), lambda i: (0, 0)),
        pl.BlockSpec((F, F * F), lambda i: (0, 0)),
        pl.BlockSpec((F, F * F), lambda i: (0, 0)),
        pl.BlockSpec((F * F, F), lambda i: (0, 0)),
        pl.BlockSpec((F, F), lambda i: (0, 0)),
    ],
    out_specs=pl.BlockSpec((BLK, 128), lambda i: (i, 0)),
    out_shape=jax.ShapeDtypeStruct((E_PAD, 128), jnp.float32),
)


def _final_body(acc_ref, nat_ref, lnw_ref, lnb_ref, o_ref):
    a = acc_ref[0] + acc_ref[1]
    summed = a[:, 0:F]
    cnt = a[:, F:F + 1]
    out = summed / jnp.maximum(cnt, 1.0) + nat_ref[...]
    mean = jnp.sum(out, axis=1, keepdims=True) * (1.0 / F)
    xc = out - mean
    var = jnp.sum(xc * xc, axis=1, keepdims=True) * (1.0 / F)
    norm = lax.rsqrt(var + 1e-5)
    o_ref[...] = xc * (norm * lnw_ref[...]) + lnb_ref[...]


_final_call = pl.pallas_call(
    _final_body,
    out_shape=jax.ShapeDtypeStruct((N_NODES, F), jnp.float32),
)


def kernel(node_attr, edge_index, edge_attr, edge_sh, W1, b1, W2, b2, ln_w, ln_b):
    gather_call, scatter_call = _sc_calls()
    eip = jnp.pad(edge_index, ((0, 0), (0, E_PAD - N_EDGES)))
    dst2 = eip[1].reshape(NCHUNK, CH)
    src2 = eip[0].reshape(NCHUNK, CH)
    xg = gather_call(dst2, node_attr)
    V = W2.reshape(F, F, F).reshape(F * F, F)
    B2 = b2.reshape(F, F)
    tpe = _dense_call(edge_attr.T, xg, edge_sh.T, W1, b1.reshape(1, F),
                      jnp.asarray(_R_np), jnp.asarray(_T_np), V, B2)
    zer = jnp.zeros((NZR, 2 * F), jnp.float32)
    acc = scatter_call(src2, tpe, zer)
    out = _final_call(acc, node_attr, ln_w.reshape(1, F), ln_b.reshape(1, F))
    return out


# speedup vs baseline: 6.3118x; 1.0860x over previous
import functools

import jax
import jax.numpy as jnp
import numpy as _np
from jax import lax
from jax.experimental import pallas as pl
from jax.experimental.pallas import tpu as pltpu
from jax.experimental.pallas import tpu_sc as plsc

N_NODES = 10000
N_EDGES = 320000
F = 16
CH = 128
NCHUNK = 2560
E_PAD = NCHUNK * CH
NC = 2
NSUB = 16
NHALF = NCHUNK // 2
E_HALF = NHALF * CH
CPW = NHALF // (NC * NSUB)
GROUP = 8
NZR = 624

_SC_PARAMS = pltpu.CompilerParams(use_tc_tiling_on_sc=False)


def _make_gather_body(half):
    def body(dst_hbm, nattr_hbm, xg_hbm, idx_v, rows_v, sem_g, sem_c):
        w = lax.axis_index("s") * NC + lax.axis_index("c")
        lbase = w * CPW
        pltpu.sync_copy(dst_hbm.at[pl.ds(half * NHALF + lbase, CPW)], idx_v)
        ng = CPW // GROUP
        outs = [None] * ng
        for g in range(ng):
            b = g % 2
            if g >= 2:
                outs[g - 2].wait()
            cps = []
            for j in range(GROUP):
                t = g * GROUP + j
                cps.append(pltpu.async_copy(
                    nattr_hbm.at[idx_v.at[t]],
                    rows_v.at[b, pl.ds(j * CH, CH)], sem_g))
            for cp in cps:
                cp.wait()
            outs[g] = pltpu.async_copy(
                rows_v.at[b],
                xg_hbm.at[pl.ds((lbase + g * GROUP) * CH, GROUP * CH),
                          pl.ds(0, F)],
                sem_c)
        outs[ng - 2].wait()
        outs[ng - 1].wait()
    return body


def _make_scatter_body(half):
    def body(src_hbm, tpe_hbm, zer_hbm, acc_hbm,
             idx_v, rows_v, zbuf_v, obuf_v, sem_l, sem_a, shared):
        c = lax.axis_index("c")
        s = lax.axis_index("s")
        w = s * NC + c
        lbase = w * CPW
        pltpu.sync_copy(src_hbm.at[pl.ds(half * NHALF + lbase, CPW)], idx_v)
        pltpu.sync_copy(zer_hbm, zbuf_v)
        pltpu.sync_copy(zbuf_v.at[pl.ds(0, NZR)],
                        shared.at[pl.ds(s * NZR, NZR)])

        @pl.when(s == 0)
        def _():
            pltpu.sync_copy(zbuf_v.at[pl.ds(0, 16)],
                            shared.at[pl.ds(16 * NZR, 16)])

        plsc.subcore_barrier()

        def load(t, b):
            return pltpu.async_copy(
                tpe_hbm.at[pl.ds((lbase + t) * CH, CH), pl.ds(0, 2 * F)],
                rows_v.at[b], sem_l)

        lds = [None, None]
        adds = [None, None]
        lds[0] = load(0, 0)
        for t in range(CPW):
            b = t % 2
            lds[b].wait()
            if adds[1 - b] is not None:
                adds[1 - b].wait()
            if t + 1 < CPW:
                lds[1 - b] = load(t + 1, 1 - b)
            adds[b] = pltpu.async_copy(rows_v.at[b], shared.at[idx_v.at[t]],
                                       sem_a, add=True)
        adds[(CPW - 1) % 2].wait()
        plsc.subcore_barrier()
        pltpu.sync_copy(shared.at[pl.ds(s * NZR, NZR)], obuf_v)
        pltpu.sync_copy(obuf_v, acc_hbm.at[c, pl.ds(s * NZR, NZR)])

        @pl.when(s == 0)
        def _():
            pltpu.sync_copy(shared.at[pl.ds(16 * NZR, 16)],
                            obuf_v.at[pl.ds(0, 16)])
            pltpu.sync_copy(obuf_v.at[pl.ds(0, 16)],
                            acc_hbm.at[c, pl.ds(16 * NZR, 16)])
    return body


@functools.lru_cache(maxsize=None)
def _sc_calls():
    mesh = plsc.VectorSubcoreMesh(
        core_axis_name="c", subcore_axis_name="s",
        num_cores=NC, num_subcores=NSUB)
    gathers, scatters = [], []
    for half in range(2):
        gathers.append(pl.kernel(
            _make_gather_body(half),
            out_type=jax.ShapeDtypeStruct((E_HALF, 128), jnp.float32),
            mesh=mesh,
            compiler_params=_SC_PARAMS,
            scratch_types=[
                pltpu.VMEM((CPW, CH), jnp.int32),
                pltpu.VMEM((2, GROUP * CH, F), jnp.float32),
                pltpu.SemaphoreType.DMA,
                pltpu.SemaphoreType.DMA,
            ],
        ))
        scatters.append(pl.kernel(
            _make_scatter_body(half),
            out_type=jax.ShapeDtypeStruct((NC, N_NODES, 2 * F), jnp.float32),
            mesh=mesh,
            compiler_params=_SC_PARAMS,
            scratch_types=[
                pltpu.VMEM((CPW, CH), jnp.int32),
                pltpu.VMEM((2, CH, 2 * F), jnp.float32),
                pltpu.VMEM((NZR, 2 * F), jnp.float32),
                pltpu.VMEM((NZR, 2 * F), jnp.float32),
                pltpu.SemaphoreType.DMA,
                pltpu.SemaphoreType.DMA,
                pltpu.VMEM_SHARED((N_NODES, 2 * F), jnp.float32),
            ],
        ))
    return gathers, scatters


BLK = 2560
NBLK_H = E_HALF // BLK
_NBLK_REAL = N_EDGES // BLK

_R_np = _np.kron(_np.eye(F, dtype=_np.float32), _np.ones((1, F), _np.float32))
_T_np = _np.kron(_np.ones((1, F), _np.float32), _np.eye(F, dtype=_np.float32))


def _mm(a, b):
    return lax.dot_general(a, b, (((1,), (0,)), ((), ())),
                           preferred_element_type=jnp.float32)


def _make_dense_body(half):
    def body(ea_ref, xg_ref, sh_ref, W1_ref, b1_ref, R_ref, T_ref, V_ref,
             B2_ref, o_ref):
        i = pl.program_id(0)
        ea = jnp.transpose(ea_ref[...], (1, 0))
        sh = jnp.transpose(sh_ref[...], (1, 0))
        h = jnp.maximum(_mm(ea, W1_ref[...]) + b1_ref[...], 0.0)
        xs = xg_ref[:, 0:F] * sh
        z = _mm(h, R_ref[...]) * _mm(xs, T_ref[...])
        tp = (_mm(z, V_ref[...]) + _mm(xs, B2_ref[...])) * 0.25
        validf = (((half * NBLK_H + i) * BLK
                   + lax.broadcasted_iota(jnp.int32, (BLK, 1), 0)
                   < N_EDGES)).astype(jnp.float32)
        col0 = (lax.broadcasted_iota(jnp.int32, (BLK, F), 1) == 0)
        o_ref[:, 0:F] = tp * validf
        o_ref[:, F:2 * F] = validf * col0.astype(jnp.float32)
        o_ref[:, 2 * F:128] = jnp.zeros((BLK, 128 - 2 * F), jnp.float32)
    return body


def _make_dense_call(half):
    def clampg(i):
        return jnp.minimum(half * NBLK_H + i, _NBLK_REAL - 1)

    return pl.pallas_call(
        _make_dense_body(half),
        grid=(NBLK_H,),
        in_specs=[
            pl.BlockSpec((F, BLK), lambda i: (0, clampg(i))),
            pl.BlockSpec((BLK, 128), lambda i: (i, 0)),
            pl.BlockSpec((1, BLK), lambda i: (0, clampg(i))),
            pl.BlockSpec((F, F), lambda i: (0, 0)),
            pl.BlockSpec((1, F), lambda i: (0, 0)),
            pl.BlockSpec((F, F * F), lambda i: (0, 0)),
            pl.BlockSpec((F, F * F), lambda i: (0, 0)),
            pl.BlockSpec((F * F, F), lambda i: (0, 0)),
            pl.BlockSpec((F, F), lambda i: (0, 0)),
        ],
        out_specs=pl.BlockSpec((BLK, 128), lambda i: (i, 0)),
        out_shape=jax.ShapeDtypeStruct((E_HALF, 128), jnp.float32),
    )


_dense_calls = [_make_dense_call(0), _make_dense_call(1)]


def _final_body(a0_ref, a1_ref, nat_ref, lnw_ref, lnb_ref, o_ref):
    a = a0_ref[0] + a0_ref[1] + a1_ref[0] + a1_ref[1]
    summed = a[:, 0:F]
    cnt = a[:, F:F + 1]
    out = summed / jnp.maximum(cnt, 1.0) + nat_ref[...]
    mean = jnp.sum(out, axis=1, keepdims=True) * (1.0 / F)
    xc = out - mean
    var = jnp.sum(xc * xc, axis=1, keepdims=True) * (1.0 / F)
    norm = lax.rsqrt(var + 1e-5)
    o_ref[...] = xc * (norm * lnw_ref[...]) + lnb_ref[...]


_final_call = pl.pallas_call(
    _final_body,
    out_shape=jax.ShapeDtypeStruct((N_NODES, F), jnp.float32),
)


def kernel(node_attr, edge_index, edge_attr, edge_sh, W1, b1, W2, b2, ln_w, ln_b):
    gathers, scatters = _sc_calls()
    eip = jnp.pad(edge_index, ((0, 0), (0, E_PAD - N_EDGES)))
    dst2 = eip[1].reshape(NCHUNK, CH)
    src2 = eip[0].reshape(NCHUNK, CH)
    V = W2.reshape(F, F, F).reshape(F * F, F)
    B2 = b2.reshape(F, F)
    Rc = jnp.asarray(_R_np)
    Tc = jnp.asarray(_T_np)
    zer = jnp.zeros((NZR, 2 * F), jnp.float32)
    eaT = edge_attr.T
    shT = edge_sh.T
    b1r = b1.reshape(1, F)

    xg0 = gathers[0](dst2, node_attr)
    xg1 = gathers[1](dst2, node_attr)
    tpe0 = _dense_calls[0](eaT, xg0, shT, W1, b1r, Rc, Tc, V, B2)
    tpe1 = _dense_calls[1](eaT, xg1, shT, W1, b1r, Rc, Tc, V, B2)
    acc0 = scatters[0](src2, tpe0, zer)
    acc1 = scatters[1](src2, tpe1, zer)
    out = _final_call(acc0, acc1, node_attr, ln_w.reshape(1, F),
                      ln_b.reshape(1, F))
    return out
